# baseline (speedup 1.0000x reference)
import jax
import jax.numpy as jnp
from jax import lax
from jax.experimental import pallas as pl
from jax.experimental.pallas import tpu as pltpu
from jax.experimental.pallas import tpu_sc as plsc

_NC = 2
_NS = 16
_CH = 128


def kernel(temporal_info, doy_weight):
    B, T, N, F = temporal_info.shape
    V, D = doy_weight.shape
    num_indices = B * N

    flat_ti = temporal_info.astype(jnp.int32).reshape(B * T * N * F)
    p = jnp.arange(num_indices, dtype=jnp.int32)
    word_idx = ((p // N) * (T * N * F) + (p % N) * F + (F - 1)).reshape(
        1, num_indices)

    per_w = num_indices // (_NC * _NS)
    nch = per_w // _CH

    mesh = plsc.VectorSubcoreMesh(core_axis_name="c", subcore_axis_name="s")

    @pl.kernel(
        out_type=jax.ShapeDtypeStruct((num_indices, D), doy_weight.dtype),
        mesh=mesh,
        scratch_types=[
            pltpu.VMEM_SHARED((V, D), doy_weight.dtype),
            pltpu.VMEM((per_w,), jnp.int32),
            pltpu.VMEM((per_w,), jnp.int32),
            pltpu.VMEM((2, 2 * _CH, D), doy_weight.dtype),
            pltpu.SemaphoreType.DMA,
            pltpu.SemaphoreType.DMA,
        ],
    )
    def gather_kernel(ti_hbm, widx_hbm, table_hbm, out_hbm,
                      table_s, widx_v, idx_v, rows_v, sem0, sem1):
        c = lax.axis_index("c")
        s = lax.axis_index("s")

        base = (c * _NS + s) * per_w
        pltpu.sync_copy(widx_hbm.at[0, pl.ds(base, per_w)], widx_v)

        doy_dmas = [
            pltpu.async_copy(
                ti_hbm.at[widx_v.at[pl.ds(ch * _CH, _CH)]],
                idx_v.at[pl.ds(ch * _CH, _CH)],
                sem1,
            )
            for ch in range(nch)
        ]

        @pl.when(s == 0)
        def _():
            pltpu.sync_copy(table_hbm, table_s)

        plsc.subcore_barrier()
        for dma in doy_dmas:
            dma.wait()

        sems = (sem0, sem1)
        ngrp = nch // 2
        grp_rows = 2 * _CH

        def run_group(g, b):
            for h in range(2):
                pltpu.sync_copy(
                    table_s.at[idx_v.at[pl.ds((2 * g + h) * _CH, _CH)]],
                    rows_v.at[b, pl.ds(h * _CH, _CH)],
                )
            return pltpu.async_copy(
                rows_v.at[b],
                out_hbm.at[pl.ds(base + g * grp_rows, grp_rows)],
                sems[b],
            )

        primed = [run_group(g, g) for g in range(2)]

        @pl.loop(2, ngrp, step=2)
        def _(g):
            for b in range(2):
                pltpu.make_async_copy(
                    rows_v.at[b],
                    out_hbm.at[pl.ds(base, grp_rows)],
                    sems[b],
                ).wait()
                run_group(g + b, b)

        for b in range(2):
            primed[b].wait()

    return gather_kernel(flat_ti, word_idx, doy_weight).reshape(B, N, D)
